# baseline (speedup 1.0000x reference)
import functools

import jax
import jax.numpy as jnp
from jax import lax
from jax.experimental import pallas as pl
from jax.experimental.pallas import tpu as pltpu
from jax.experimental.pallas import tpu_sc as plsc

_EPS = 1e-6
_NEG = -1e30


def _silu(g):
    return g * (1.0 / (1.0 + jnp.exp(-g)))


def _shared_body(x_ref, norm_ref, w1_ref, w3_ref, w2_ref, out_ref, *, ns):
    xv = x_ref[...]
    ms = jnp.mean(xv * xv, axis=1, keepdims=True)
    xn = xv * jax.lax.rsqrt(ms + _EPS)
    acc = None
    for i in range(ns):
        xni = (xn * norm_ref[i:i + 1, :]).astype(jnp.bfloat16)
        g = jnp.dot(xni, w1_ref[i], preferred_element_type=jnp.float32)
        v = jnp.dot(xni, w3_ref[i], preferred_element_type=jnp.float32)
        h = (_silu(g) * v).astype(jnp.bfloat16)
        y = jnp.dot(h, w2_ref[i], preferred_element_type=jnp.float32)
        acc = y if acc is None else acc + y
    out_ref[...] = xv + acc * (1.0 / ns)


def _shared_call(x_flat, norm_p, w1c, w3c, w2c, tm):
    t, d = x_flat.shape
    ns, _, fs = w1c.shape
    body = functools.partial(_shared_body, ns=ns)
    return pl.pallas_call(
        body,
        grid=(t // tm,),
        in_specs=[
            pl.BlockSpec((tm, d), lambda m: (m, 0)),
            pl.BlockSpec((8, d), lambda m: (0, 0)),
            pl.BlockSpec((ns, d, fs), lambda m: (0, 0, 0)),
            pl.BlockSpec((ns, d, fs), lambda m: (0, 0, 0)),
            pl.BlockSpec((ns, fs, d), lambda m: (0, 0, 0)),
        ],
        out_specs=pl.BlockSpec((tm, d), lambda m: (m, 0)),
        out_shape=jax.ShapeDtypeStruct((t, d), jnp.float32),
    )(x_flat, norm_p, w1c, w3c, w2c)


def _router_body(x_ref, down_ref, up_ref, pack_ref, counts_ref, lbl_ref,
                 cnt_s, *, n_experts):
    pid = pl.program_id(0)
    ntiles = pl.num_programs(0)
    tb = x_ref.shape[0]

    @pl.when(pid == 0)
    def _():
        cnt_s[...] = jnp.zeros_like(cnt_s)

    xv = x_ref[...]
    t1 = jnp.dot(xv, down_ref[...], preferred_element_type=jnp.float32)
    logits = jnp.dot(t1, up_ref[...], preferred_element_type=jnp.float32)
    col = jax.lax.broadcasted_iota(jnp.int32, (tb, 128), 1)
    valid = col < n_experts
    lg = jnp.where(valid, logits, _NEG)
    m1 = jnp.max(lg, axis=1, keepdims=True)
    e1 = jnp.min(jnp.where((lg == m1) & valid, col, 16384), axis=1)
    lg2 = jnp.where(col == e1[:, None], _NEG, lg)
    m2 = jnp.max(lg2, axis=1, keepdims=True)
    e2 = jnp.min(jnp.where((lg2 == m2) & valid, col, 16384), axis=1)
    q = jnp.exp(m2 - m1)
    w0 = 1.0 / (1.0 + q)
    w1 = q * w0

    m_one = jnp.concatenate(
        [(col == e1[:, None]).astype(jnp.float32),
         (col == e2[:, None]).astype(jnp.float32)], axis=0)
    ii = jax.lax.broadcasted_iota(jnp.int32, (2 * tb, 2 * tb), 0)
    jj = jax.lax.broadcasted_iota(jnp.int32, (2 * tb, 2 * tb), 1)
    tri = (ii > jj).astype(jnp.float32)
    rank = jnp.dot(tri, m_one, preferred_element_type=jnp.float32)
    cs = cnt_s[...]
    prev = cs[0:1, :]
    within = jnp.sum(rank * m_one, axis=1) + jnp.sum(m_one * prev, axis=1)
    new_row = prev + jnp.sum(m_one, axis=0, keepdims=True)
    cnt_s[...] = jnp.broadcast_to(new_row, cnt_s.shape)
    counts_ref[...] = jnp.broadcast_to(new_row, counts_ref.shape)

    pack_ref[...] = jnp.concatenate(
        [e1[:, None].astype(jnp.float32), e2[:, None].astype(jnp.float32),
         within[:tb, None], within[tb:, None], w0, w1,
         jnp.zeros((tb, 2), jnp.float32)], axis=1)

    @pl.when(pid == ntiles - 1)
    def _():
        row = new_row[0, :]
        msk = (jax.lax.iota(jnp.int32, 128) < n_experts).astype(jnp.float32)
        mean = jnp.sum(row * msk) / n_experts
        var = jnp.sum(((row - mean) * msk) ** 2) / (n_experts - 1)
        lbl_ref[...] = jnp.full_like(lbl_ref, var)


def _router_call(x_flat, down_p, up_p, tb, n_experts):
    t, d = x_flat.shape
    body = functools.partial(_router_body, n_experts=n_experts)
    return pl.pallas_call(
        body,
        grid=(t // tb,),
        in_specs=[
            pl.BlockSpec((tb, d), lambda m: (m, 0)),
            pl.BlockSpec((d, 128), lambda m: (0, 0)),
            pl.BlockSpec((128, 128), lambda m: (0, 0)),
        ],
        out_specs=[
            pl.BlockSpec((tb, 8), lambda m: (m, 0)),
            pl.BlockSpec((8, 128), lambda m: (0, 0)),
            pl.BlockSpec((8, 128), lambda m: (0, 0)),
        ],
        out_shape=[
            jax.ShapeDtypeStruct((t, 8), jnp.float32),
            jax.ShapeDtypeStruct((8, 128), jnp.float32),
            jax.ShapeDtypeStruct((8, 128), jnp.float32),
        ],
        scratch_shapes=[pltpu.VMEM((8, 128), jnp.float32)],
    )(x_flat, down_p, up_p)


def _gmm_body(meta_ref, xs_ref, ws_ref, w1_ref, w3_ref, w2_ref, ys_ref,
              w1_s, w3_s, w2_s, w1_b, w3_b, w2_b, s1, s3, s2, *, te):
    j = pl.program_id(0)
    start = meta_ref[2, j]
    end = meta_ref[3, j]
    first = meta_ref[4, j]
    tile = meta_ref[0, j]
    slot = meta_ref[1, j]

    def issue(sl, e):
        pltpu.make_async_copy(w1_ref.at[e], w1_s.at[sl], s1.at[sl]).start()
        pltpu.make_async_copy(w3_ref.at[e], w3_s.at[sl], s3.at[sl]).start()
        pltpu.make_async_copy(w2_ref.at[e], w2_s.at[sl], s2.at[sl]).start()

    def wait(sl):
        pltpu.make_async_copy(w1_ref.at[0], w1_s.at[sl], s1.at[sl]).wait()
        pltpu.make_async_copy(w3_ref.at[0], w3_s.at[sl], s3.at[sl]).wait()
        pltpu.make_async_copy(w2_ref.at[0], w2_s.at[sl], s2.at[sl]).wait()

    @pl.when(j == 0)
    def _():
        issue(slot, meta_ref[6, j])

    @pl.when(meta_ref[5, j] == 1)
    def _():
        wait(slot)
        w1_b[...] = w1_s[slot].astype(jnp.bfloat16)
        w3_b[...] = w3_s[slot].astype(jnp.bfloat16)
        w2_b[...] = w2_s[slot].astype(jnp.bfloat16)
        nxt = meta_ref[7, j]

        @pl.when(nxt >= 0)
        def _():
            issue(1 - slot, nxt)

    xv = xs_ref[...].astype(jnp.bfloat16)
    g = jnp.dot(xv, w1_b[...], preferred_element_type=jnp.float32)
    v = jnp.dot(xv, w3_b[...], preferred_element_type=jnp.float32)
    h = (_silu(g) * v).astype(jnp.bfloat16)
    y = jnp.dot(h, w2_b[...], preferred_element_type=jnp.float32)
    y = y * ws_ref[:, 0:1]
    rglob = jax.lax.broadcasted_iota(jnp.int32, (te, 1), 0) + tile * te
    y = jnp.where((rglob >= start) & (rglob < end), y, 0.0)

    @pl.when(first == 1)
    def _():
        ys_ref[...] = y

    @pl.when(first == 0)
    def _():
        ys_ref[...] += y


def _gmm_call(xs, ws, w1b, w3b, w2b, meta, nwi, te):
    n, d = xs.shape
    f = w1b.shape[2]
    body = functools.partial(_gmm_body, te=te)
    grid_spec = pltpu.PrefetchScalarGridSpec(
        num_scalar_prefetch=1,
        grid=(nwi,),
        in_specs=[
            pl.BlockSpec((te, d), lambda j, m: (m[0, j], 0)),
            pl.BlockSpec((te, 128), lambda j, m: (m[0, j], 0)),
            pl.BlockSpec(memory_space=pltpu.MemorySpace.HBM),
            pl.BlockSpec(memory_space=pltpu.MemorySpace.HBM),
            pl.BlockSpec(memory_space=pltpu.MemorySpace.HBM),
        ],
        out_specs=pl.BlockSpec((te, d), lambda j, m: (m[0, j], 0)),
        scratch_shapes=[pltpu.VMEM((2, d, f), jnp.float32),
                        pltpu.VMEM((2, d, f), jnp.float32),
                        pltpu.VMEM((2, f, d), jnp.float32),
                        pltpu.VMEM((d, f), jnp.bfloat16),
                        pltpu.VMEM((d, f), jnp.bfloat16),
                        pltpu.VMEM((f, d), jnp.bfloat16),
                        pltpu.SemaphoreType.DMA((2,)),
                        pltpu.SemaphoreType.DMA((2,)),
                        pltpu.SemaphoreType.DMA((2,))],
    )
    return pl.pallas_call(
        body,
        grid_spec=grid_spec,
        out_shape=jax.ShapeDtypeStruct((n, d), jnp.float32),
    )(meta, xs, ws, w1b, w3b, w2b)


def _gmm_metadata(counts, n_pairs, te, n_experts):
    offs = jnp.concatenate([jnp.zeros((1,), jnp.int32), jnp.cumsum(counts)])
    lo, hi = offs[:n_experts], offs[1:]
    t0 = lo // te
    t1 = jnp.where(hi > lo, (hi + te - 1) // te, t0)
    cnt = t1 - t0
    cum = jnp.cumsum(cnt)
    nwi = n_pairs // te + n_experts - 1
    jidx = jnp.arange(nwi, dtype=jnp.int32)
    ej = jnp.minimum(jnp.sum(cum[:, None] <= jidx[None, :], axis=0),
                     n_experts - 1).astype(jnp.int32)
    base = jnp.concatenate([jnp.zeros((1,), jnp.int32), cum])[ej]
    tile_j = t0[ej] + (jidx - base)
    valid = jidx < cum[n_experts - 1]
    last_tile = n_pairs // te - 1
    tile_j = jnp.where(valid, tile_j, last_tile)
    start_j = jnp.where(valid, jnp.maximum(lo[ej], tile_j * te), 0)
    end_j = jnp.where(valid, jnp.minimum(hi[ej], (tile_j + 1) * te), 0)
    first_j = jnp.concatenate(
        [jnp.ones((1,), jnp.int32),
         (tile_j[1:] != tile_j[:-1]).astype(jnp.int32)])
    first_j = jnp.where(valid, first_j, 0)
    echg_j = jnp.concatenate(
        [jnp.ones((1,), jnp.int32),
         (ej[1:] != ej[:-1]).astype(jnp.int32)])
    slot_j = (jnp.cumsum(echg_j) - 1) % 2
    big = jnp.int32(nwi + 1)
    start_pos = jnp.where(echg_j == 1, jidx, big)
    sfx = lax.cummin(jnp.flip(start_pos))
    next_pos = jnp.concatenate([jnp.flip(sfx)[1:], jnp.full((1,), big)])
    next_e = jnp.where(next_pos < nwi, ej[jnp.minimum(next_pos, nwi - 1)], -1)
    meta = jnp.stack([tile_j, slot_j, start_j, end_j, first_j, echg_j,
                      ej, next_e], axis=0)
    meta = jnp.pad(meta, ((0, 0), (0, 64 - nwi))).astype(jnp.int32)
    return meta, offs, nwi


def _make_dispatch(t, d, npairs):
    mesh = plsc.VectorSubcoreMesh(core_axis_name="c", subcore_axis_name="s")
    nw = 32
    pw = npairs // nw

    @functools.partial(
        pl.kernel, mesh=mesh,
        compiler_params=pltpu.CompilerParams(needs_layout_passes=False),
        out_type=[jax.ShapeDtypeStruct((npairs, d), jnp.float32),
                  jax.ShapeDtypeStruct((npairs,), jnp.int32),
                  jax.ShapeDtypeStruct((npairs, 128), jnp.float32)],
        scratch_types=[pltpu.VMEM((pw,), jnp.int32),
                       pltpu.VMEM((pw,), jnp.int32),
                       pltpu.VMEM((pw,), jnp.int32),
                       pltpu.VMEM((16,), jnp.int32),
                       pltpu.VMEM((pw, d), jnp.float32),
                       pltpu.VMEM((pw,), jnp.float32),
                       pltpu.VMEM((pw, 128), jnp.float32),
                       pltpu.SemaphoreType.DMA,
                       pltpu.SemaphoreType.DMA],
    )
    def dispatch(x_hbm, ew_hbm, wr_hbm, off_hbm, w_hbm,
                 xs_out, dest_out, ws_out,
                 e_v, wr_v, dest_v, off_v, rows_v, w_v, wbuf_v, sem, sem2):
        wid = lax.axis_index("s") * 2 + lax.axis_index("c")
        base = wid * pw
        tbase = (wid % 16) * pw
        pltpu.sync_copy(ew_hbm.at[pl.ds(base, pw)], e_v)
        pltpu.sync_copy(wr_hbm.at[pl.ds(base, pw)], wr_v)
        pltpu.sync_copy(w_hbm.at[pl.ds(base, pw)], w_v)
        pltpu.sync_copy(off_hbm, off_v)
        zero16 = jnp.zeros((16,), jnp.int32)
        for c in range(pw // 16):
            sl = pl.ds(c * 16, 16)
            off = plsc.load_gather(off_v, [e_v[sl]])
            dest_v[sl] = off + wr_v[sl]
            rix = lax.iota(jnp.int32, 16) + c * 16
            plsc.store_scatter(wbuf_v, [rix, zero16], w_v[sl])
        pltpu.sync_copy(dest_v, dest_out.at[pl.ds(base, pw)])
        pltpu.sync_copy(x_hbm.at[pl.ds(tbase, pw)], rows_v)
        cp1 = pltpu.async_copy(rows_v, xs_out.at[dest_v], sem)
        cp2 = pltpu.async_copy(wbuf_v, ws_out.at[dest_v], sem2)
        cp1.wait()
        cp2.wait()

    return dispatch


def _make_combine(t, d, npairs):
    mesh = plsc.VectorSubcoreMesh(core_axis_name="c", subcore_axis_name="s")
    nw = 32
    tw = t // nw
    hc = tw // 2

    @functools.partial(
        pl.kernel, mesh=mesh,
        compiler_params=pltpu.CompilerParams(needs_layout_passes=False),
        out_type=jax.ShapeDtypeStruct((t, d), jnp.float32),
        scratch_types=[pltpu.VMEM((hc,), jnp.int32),
                       pltpu.VMEM((hc,), jnp.int32),
                       pltpu.VMEM((hc, d), jnp.float32),
                       pltpu.VMEM((hc, d), jnp.float32),
                       pltpu.VMEM((hc, d), jnp.float32),
                       pltpu.SemaphoreType.DMA,
                       pltpu.SemaphoreType.DMA],
    )
    def combine(ys_hbm, sh_hbm, dest_hbm, out_hbm,
                d0_v, d1_v, r0_v, r1_v, acc_v, sem, sem2):
        wid = lax.axis_index("s") * 2 + lax.axis_index("c")
        for h in range(2):
            base = wid * tw + h * hc
            pltpu.sync_copy(dest_hbm.at[pl.ds(base, hc)], d0_v)
            pltpu.sync_copy(dest_hbm.at[pl.ds(t + base, hc)], d1_v)
            g0 = pltpu.async_copy(ys_hbm.at[d0_v], r0_v, sem)
            g1 = pltpu.async_copy(ys_hbm.at[d1_v], r1_v, sem2)
            pltpu.sync_copy(sh_hbm.at[pl.ds(base, hc)], acc_v)
            g0.wait()
            g1.wait()

            def tok(i, _):
                for c in range(d // 16):
                    sl = pl.ds(c * 16, 16)
                    acc_v[i, sl] = acc_v[i, sl] + r0_v[i, sl] + r1_v[i, sl]
                return 0

            lax.fori_loop(0, hc, tok, 0)
            pltpu.sync_copy(acc_v, out_hbm.at[pl.ds(base, hc)])

    return combine


def kernel(x, sh_norm, sh_w1, sh_w2, sh_w3, r_w1, r_w2, r_w3,
           router_down, router_up):
    bsz, seq, d = x.shape
    t = bsz * seq
    ns, _, fs = sh_w1.shape
    ne, _, fr = r_w1.shape
    k = 2
    x_flat = x.reshape(t, d)

    w1c = sh_w1.astype(jnp.bfloat16)
    w3c = sh_w3.astype(jnp.bfloat16)
    w2c = sh_w2.astype(jnp.bfloat16)
    norm_p = jnp.pad(sh_norm, ((0, 8 - ns), (0, 0)))
    down_p = jnp.pad(router_down, ((0, 0), (0, 128 - router_up.shape[0])))
    up_p = jnp.pad(router_up, ((0, 128 - router_up.shape[0]),
                               (0, 128 - router_up.shape[1])))

    shared = _shared_call(x_flat, norm_p, w1c, w3c, w2c, tm=256)
    pack, counts_w, lbl_w = _router_call(x_flat, down_p, up_p, tb=256,
                                         n_experts=ne)
    counts = counts_w[0, :ne].astype(jnp.int32)
    lbl = lbl_w[0, 0]

    ew = jnp.concatenate([pack[:, 0], pack[:, 1]]).astype(jnp.int32)
    wrw = jnp.concatenate([pack[:, 2], pack[:, 3]]).astype(jnp.int32)
    wpack = jnp.concatenate([pack[:, 4], pack[:, 5]])

    te = 512
    meta, offs, nwi = _gmm_metadata(counts, t * k, te, ne)
    offs16 = jnp.pad(offs[:ne], (0, 16 - ne))

    xs, dest, ws = _make_dispatch(t, d, t * k)(x_flat, ew, wrw, offs16, wpack)

    ys = _gmm_call(xs, ws, r_w1, r_w3, r_w2, meta, nwi, te)

    out = _make_combine(t, d, t * k)(ys, shared, dest)

    return (out.reshape(bsz, seq, d), lbl,
            jnp.float32(0.0), jnp.float32(0.0))


# speedup vs baseline: 1.0363x; 1.0363x over previous
import functools

import jax
import jax.numpy as jnp
from jax import lax
from jax.experimental import pallas as pl
from jax.experimental.pallas import tpu as pltpu
from jax.experimental.pallas import tpu_sc as plsc

_EPS = 1e-6
_NEG = -1e30


def _silu(g):
    return g * (1.0 / (1.0 + jnp.exp(-g)))


def _shared_body(x_ref, norm_ref, w1_ref, w3_ref, w2_ref, out_ref, *, ns):
    xv = x_ref[...]
    ms = jnp.mean(xv * xv, axis=1, keepdims=True)
    xn = xv * jax.lax.rsqrt(ms + _EPS)
    acc = None
    for i in range(ns):
        xni = (xn * norm_ref[i:i + 1, :]).astype(jnp.bfloat16)
        g = jnp.dot(xni, w1_ref[i], preferred_element_type=jnp.float32)
        v = jnp.dot(xni, w3_ref[i], preferred_element_type=jnp.float32)
        h = (_silu(g) * v).astype(jnp.bfloat16)
        y = jnp.dot(h, w2_ref[i], preferred_element_type=jnp.float32)
        acc = y if acc is None else acc + y
    out_ref[...] = xv + acc * (1.0 / ns)


def _shared_call(x_flat, norm_p, w1c, w3c, w2c, tm):
    t, d = x_flat.shape
    ns, _, fs = w1c.shape
    body = functools.partial(_shared_body, ns=ns)
    return pl.pallas_call(
        body,
        grid=(t // tm,),
        in_specs=[
            pl.BlockSpec((tm, d), lambda m: (m, 0)),
            pl.BlockSpec((8, d), lambda m: (0, 0)),
            pl.BlockSpec((ns, d, fs), lambda m: (0, 0, 0)),
            pl.BlockSpec((ns, d, fs), lambda m: (0, 0, 0)),
            pl.BlockSpec((ns, fs, d), lambda m: (0, 0, 0)),
        ],
        out_specs=pl.BlockSpec((tm, d), lambda m: (m, 0)),
        out_shape=jax.ShapeDtypeStruct((t, d), jnp.float32),
    )(x_flat, norm_p, w1c, w3c, w2c)


def _router_body(x_ref, down_ref, up_ref, pack_ref, counts_ref, lbl_ref,
                 cnt_s, *, n_experts):
    pid = pl.program_id(0)
    ntiles = pl.num_programs(0)
    tb = x_ref.shape[0]

    @pl.when(pid == 0)
    def _():
        cnt_s[...] = jnp.zeros_like(cnt_s)

    xv = x_ref[...]
    t1 = jnp.dot(xv, down_ref[...], preferred_element_type=jnp.float32)
    logits = jnp.dot(t1, up_ref[...], preferred_element_type=jnp.float32)
    col = jax.lax.broadcasted_iota(jnp.int32, (tb, 128), 1)
    valid = col < n_experts
    lg = jnp.where(valid, logits, _NEG)
    m1 = jnp.max(lg, axis=1, keepdims=True)
    e1 = jnp.min(jnp.where((lg == m1) & valid, col, 16384), axis=1)
    lg2 = jnp.where(col == e1[:, None], _NEG, lg)
    m2 = jnp.max(lg2, axis=1, keepdims=True)
    e2 = jnp.min(jnp.where((lg2 == m2) & valid, col, 16384), axis=1)
    q = jnp.exp(m2 - m1)
    w0 = 1.0 / (1.0 + q)
    w1 = q * w0

    m_one = jnp.concatenate(
        [(col == e1[:, None]).astype(jnp.float32),
         (col == e2[:, None]).astype(jnp.float32)], axis=0)
    ii = jax.lax.broadcasted_iota(jnp.int32, (2 * tb, 2 * tb), 0)
    jj = jax.lax.broadcasted_iota(jnp.int32, (2 * tb, 2 * tb), 1)
    tri = (ii > jj).astype(jnp.float32)
    rank = jnp.dot(tri, m_one, preferred_element_type=jnp.float32)
    cs = cnt_s[...]
    prev = cs[0:1, :]
    within = jnp.sum(rank * m_one, axis=1) + jnp.sum(m_one * prev, axis=1)
    new_row = prev + jnp.sum(m_one, axis=0, keepdims=True)
    cnt_s[...] = jnp.broadcast_to(new_row, cnt_s.shape)
    counts_ref[...] = jnp.broadcast_to(new_row, counts_ref.shape)

    pack_ref[...] = jnp.concatenate(
        [e1[:, None].astype(jnp.float32), e2[:, None].astype(jnp.float32),
         within[:tb, None], within[tb:, None], w0, w1,
         jnp.zeros((tb, 2), jnp.float32)], axis=1)

    @pl.when(pid == ntiles - 1)
    def _():
        row = new_row[0, :]
        msk = (jax.lax.iota(jnp.int32, 128) < n_experts).astype(jnp.float32)
        mean = jnp.sum(row * msk) / n_experts
        var = jnp.sum(((row - mean) * msk) ** 2) / (n_experts - 1)
        lbl_ref[...] = jnp.full_like(lbl_ref, var)


def _router_call(x_flat, down_p, up_p, tb, n_experts):
    t, d = x_flat.shape
    body = functools.partial(_router_body, n_experts=n_experts)
    return pl.pallas_call(
        body,
        grid=(t // tb,),
        in_specs=[
            pl.BlockSpec((tb, d), lambda m: (m, 0)),
            pl.BlockSpec((d, 128), lambda m: (0, 0)),
            pl.BlockSpec((128, 128), lambda m: (0, 0)),
        ],
        out_specs=[
            pl.BlockSpec((tb, 8), lambda m: (m, 0)),
            pl.BlockSpec((8, 128), lambda m: (0, 0)),
            pl.BlockSpec((8, 128), lambda m: (0, 0)),
        ],
        out_shape=[
            jax.ShapeDtypeStruct((t, 8), jnp.float32),
            jax.ShapeDtypeStruct((8, 128), jnp.float32),
            jax.ShapeDtypeStruct((8, 128), jnp.float32),
        ],
        scratch_shapes=[pltpu.VMEM((8, 128), jnp.float32)],
    )(x_flat, down_p, up_p)


def _gmm_body(meta_ref, xs_ref, ws_ref, w1_ref, w3_ref, w2_ref, ys_ref,
              w1_s, w3_s, w2_s, w1_b, w3_b, w2_b, s1, s3, s2, *, te):
    j = pl.program_id(0)
    start = meta_ref[2, j]
    end = meta_ref[3, j]
    first = meta_ref[4, j]
    tile = meta_ref[0, j]
    slot = meta_ref[1, j]

    def issue(sl, e):
        pltpu.make_async_copy(w1_ref.at[e], w1_s.at[sl], s1.at[sl]).start()
        pltpu.make_async_copy(w3_ref.at[e], w3_s.at[sl], s3.at[sl]).start()
        pltpu.make_async_copy(w2_ref.at[e], w2_s.at[sl], s2.at[sl]).start()

    def wait(sl):
        pltpu.make_async_copy(w1_ref.at[0], w1_s.at[sl], s1.at[sl]).wait()
        pltpu.make_async_copy(w3_ref.at[0], w3_s.at[sl], s3.at[sl]).wait()
        pltpu.make_async_copy(w2_ref.at[0], w2_s.at[sl], s2.at[sl]).wait()

    @pl.when(j == 0)
    def _():
        issue(slot, meta_ref[6, j])

    @pl.when(meta_ref[5, j] == 1)
    def _():
        wait(slot)
        w1_b[...] = w1_s[slot].astype(jnp.bfloat16)
        w3_b[...] = w3_s[slot].astype(jnp.bfloat16)
        w2_b[...] = w2_s[slot].astype(jnp.bfloat16)
        nxt = meta_ref[7, j]

        @pl.when(nxt >= 0)
        def _():
            issue(1 - slot, nxt)

    xv = xs_ref[...].astype(jnp.bfloat16)
    g = jnp.dot(xv, w1_b[...], preferred_element_type=jnp.float32)
    v = jnp.dot(xv, w3_b[...], preferred_element_type=jnp.float32)
    h = (_silu(g) * v).astype(jnp.bfloat16)
    y = jnp.dot(h, w2_b[...], preferred_element_type=jnp.float32)
    y = y * ws_ref[:, 0:1]
    rglob = jax.lax.broadcasted_iota(jnp.int32, (te, 1), 0) + tile * te
    y = jnp.where((rglob >= start) & (rglob < end), y, 0.0)

    @pl.when(first == 1)
    def _():
        ys_ref[...] = y

    @pl.when(first == 0)
    def _():
        ys_ref[...] += y


def _gmm_call(xs, ws, w1b, w3b, w2b, meta, nwi, te):
    n, d = xs.shape
    f = w1b.shape[2]
    body = functools.partial(_gmm_body, te=te)
    grid_spec = pltpu.PrefetchScalarGridSpec(
        num_scalar_prefetch=1,
        grid=(nwi,),
        in_specs=[
            pl.BlockSpec((te, d), lambda j, m: (m[0, j], 0)),
            pl.BlockSpec((te, 128), lambda j, m: (m[0, j], 0)),
            pl.BlockSpec(memory_space=pltpu.MemorySpace.HBM),
            pl.BlockSpec(memory_space=pltpu.MemorySpace.HBM),
            pl.BlockSpec(memory_space=pltpu.MemorySpace.HBM),
        ],
        out_specs=pl.BlockSpec((te, d), lambda j, m: (m[0, j], 0)),
        scratch_shapes=[pltpu.VMEM((2, d, f), jnp.float32),
                        pltpu.VMEM((2, d, f), jnp.float32),
                        pltpu.VMEM((2, f, d), jnp.float32),
                        pltpu.VMEM((d, f), jnp.bfloat16),
                        pltpu.VMEM((d, f), jnp.bfloat16),
                        pltpu.VMEM((f, d), jnp.bfloat16),
                        pltpu.SemaphoreType.DMA((2,)),
                        pltpu.SemaphoreType.DMA((2,)),
                        pltpu.SemaphoreType.DMA((2,))],
    )
    return pl.pallas_call(
        body,
        grid_spec=grid_spec,
        out_shape=jax.ShapeDtypeStruct((n, d), jnp.float32),
    )(meta, xs, ws, w1b, w3b, w2b)


def _gmm_metadata(counts, n_pairs, te, n_experts):
    offs = jnp.concatenate([jnp.zeros((1,), jnp.int32), jnp.cumsum(counts)])
    lo, hi = offs[:n_experts], offs[1:]
    t0 = lo // te
    t1 = jnp.where(hi > lo, (hi + te - 1) // te, t0)
    cnt = t1 - t0
    cum = jnp.cumsum(cnt)
    nwi = n_pairs // te + n_experts - 1
    jidx = jnp.arange(nwi, dtype=jnp.int32)
    ej = jnp.minimum(jnp.sum(cum[:, None] <= jidx[None, :], axis=0),
                     n_experts - 1).astype(jnp.int32)
    base = jnp.concatenate([jnp.zeros((1,), jnp.int32), cum])[ej]
    tile_j = t0[ej] + (jidx - base)
    valid = jidx < cum[n_experts - 1]
    last_tile = n_pairs // te - 1
    tile_j = jnp.where(valid, tile_j, last_tile)
    start_j = jnp.where(valid, jnp.maximum(lo[ej], tile_j * te), 0)
    end_j = jnp.where(valid, jnp.minimum(hi[ej], (tile_j + 1) * te), 0)
    first_j = jnp.concatenate(
        [jnp.ones((1,), jnp.int32),
         (tile_j[1:] != tile_j[:-1]).astype(jnp.int32)])
    first_j = jnp.where(valid, first_j, 0)
    echg_j = jnp.concatenate(
        [jnp.ones((1,), jnp.int32),
         (ej[1:] != ej[:-1]).astype(jnp.int32)])
    slot_j = (jnp.cumsum(echg_j) - 1) % 2
    big = jnp.int32(nwi + 1)
    start_pos = jnp.where(echg_j == 1, jidx, big)
    sfx = lax.cummin(jnp.flip(start_pos))
    next_pos = jnp.concatenate([jnp.flip(sfx)[1:], jnp.full((1,), big)])
    next_e = jnp.where(next_pos < nwi, ej[jnp.minimum(next_pos, nwi - 1)], -1)
    meta = jnp.stack([tile_j, slot_j, start_j, end_j, first_j, echg_j,
                      ej, next_e], axis=0)
    meta = jnp.pad(meta, ((0, 0), (0, 64 - nwi))).astype(jnp.int32)
    return meta, offs, nwi


def _make_dispatch(t, d, npairs):
    mesh = plsc.VectorSubcoreMesh(core_axis_name="c", subcore_axis_name="s")
    nw = 32
    pw = npairs // nw

    @functools.partial(
        pl.kernel, mesh=mesh,
        compiler_params=pltpu.CompilerParams(needs_layout_passes=False),
        out_type=[jax.ShapeDtypeStruct((npairs, d), jnp.float32),
                  jax.ShapeDtypeStruct((npairs,), jnp.int32),
                  jax.ShapeDtypeStruct((npairs, 128), jnp.float32)],
        scratch_types=[pltpu.VMEM((pw,), jnp.int32),
                       pltpu.VMEM((pw,), jnp.int32),
                       pltpu.VMEM((pw,), jnp.int32),
                       pltpu.VMEM((16,), jnp.int32),
                       pltpu.VMEM((pw, d), jnp.float32),
                       pltpu.VMEM((pw,), jnp.float32),
                       pltpu.VMEM((pw, 128), jnp.float32),
                       pltpu.SemaphoreType.DMA,
                       pltpu.SemaphoreType.DMA],
    )
    def dispatch(x_hbm, ew_hbm, wr_hbm, off_hbm, w_hbm,
                 xs_out, dest_out, ws_out,
                 e_v, wr_v, dest_v, off_v, rows_v, w_v, wbuf_v, sem, sem2):
        wid = lax.axis_index("s") * 2 + lax.axis_index("c")
        base = wid * pw
        tbase = (wid % 16) * pw
        pltpu.sync_copy(ew_hbm.at[pl.ds(base, pw)], e_v)
        pltpu.sync_copy(wr_hbm.at[pl.ds(base, pw)], wr_v)
        pltpu.sync_copy(w_hbm.at[pl.ds(base, pw)], w_v)
        pltpu.sync_copy(off_hbm, off_v)
        zero16 = jnp.zeros((16,), jnp.int32)
        for c in range(pw // 16):
            sl = pl.ds(c * 16, 16)
            off = plsc.load_gather(off_v, [e_v[sl]])
            dest_v[sl] = off + wr_v[sl]
            rix = lax.iota(jnp.int32, 16) + c * 16
            plsc.store_scatter(wbuf_v, [rix, zero16], w_v[sl])
        pltpu.sync_copy(dest_v, dest_out.at[pl.ds(base, pw)])
        pltpu.sync_copy(x_hbm.at[pl.ds(tbase, pw)], rows_v)
        cp1 = pltpu.async_copy(rows_v, xs_out.at[dest_v], sem)
        cp2 = pltpu.async_copy(wbuf_v, ws_out.at[dest_v], sem2)
        cp1.wait()
        cp2.wait()

    return dispatch


def _make_combine(t, d, npairs):
    mesh = plsc.VectorSubcoreMesh(core_axis_name="c", subcore_axis_name="s")
    nw = 32
    tw = t // nw
    hc = tw // 2

    @functools.partial(
        pl.kernel, mesh=mesh,
        compiler_params=pltpu.CompilerParams(needs_layout_passes=False),
        out_type=jax.ShapeDtypeStruct((t, d), jnp.float32),
        scratch_types=[pltpu.VMEM((hc,), jnp.int32),
                       pltpu.VMEM((hc,), jnp.int32),
                       pltpu.VMEM((hc, d), jnp.float32),
                       pltpu.VMEM((hc, d), jnp.float32),
                       pltpu.VMEM((hc, d), jnp.float32),
                       pltpu.SemaphoreType.DMA,
                       pltpu.SemaphoreType.DMA],
    )
    def combine(ys_hbm, sh_hbm, dest_hbm, out_hbm,
                d0_v, d1_v, r0_v, r1_v, acc_v, sem, sem2):
        wid = lax.axis_index("s") * 2 + lax.axis_index("c")
        for h in range(2):
            base = wid * tw + h * hc
            pltpu.sync_copy(dest_hbm.at[pl.ds(base, hc)], d0_v)
            pltpu.sync_copy(dest_hbm.at[pl.ds(t + base, hc)], d1_v)
            g0 = pltpu.async_copy(ys_hbm.at[d0_v], r0_v, sem)
            g1 = pltpu.async_copy(ys_hbm.at[d1_v], r1_v, sem2)
            pltpu.sync_copy(sh_hbm.at[pl.ds(base, hc)], acc_v)
            g0.wait()
            g1.wait()

            def tok(i, _):
                for c in range(d // 16):
                    sl = pl.ds(c * 16, 16)
                    acc_v[i, sl] = acc_v[i, sl] + r0_v[i, sl] + r1_v[i, sl]
                return 0

            lax.fori_loop(0, hc, tok, 0)
            pltpu.sync_copy(acc_v, out_hbm.at[pl.ds(base, hc)])

    return combine


def kernel(x, sh_norm, sh_w1, sh_w2, sh_w3, r_w1, r_w2, r_w3,
           router_down, router_up):
    bsz, seq, d = x.shape
    t = bsz * seq
    ns, _, fs = sh_w1.shape
    ne, _, fr = r_w1.shape
    k = 2
    x_flat = x.reshape(t, d)

    w1c = sh_w1.astype(jnp.bfloat16)
    w3c = sh_w3.astype(jnp.bfloat16)
    w2c = sh_w2.astype(jnp.bfloat16)
    norm_p = jnp.pad(sh_norm, ((0, 8 - ns), (0, 0)))
    down_p = jnp.pad(router_down, ((0, 0), (0, 128 - router_up.shape[0])))
    up_p = jnp.pad(router_up, ((0, 128 - router_up.shape[0]),
                               (0, 128 - router_up.shape[1])))

    shared = _shared_call(x_flat, norm_p, w1c, w3c, w2c, tm=256)
    pack, counts_w, lbl_w = _router_call(x_flat, down_p, up_p, tb=256,
                                         n_experts=ne)
    counts = counts_w[0, :ne].astype(jnp.int32)
    lbl = lbl_w[0, 0]

    ew = jnp.concatenate([pack[:, 0], pack[:, 1]]).astype(jnp.int32)
    wrw = jnp.concatenate([pack[:, 2], pack[:, 3]]).astype(jnp.int32)
    wpack = jnp.concatenate([pack[:, 4], pack[:, 5]])

    te = 256
    meta, offs, nwi = _gmm_metadata(counts, t * k, te, ne)
    offs16 = jnp.pad(offs[:ne], (0, 16 - ne))

    xs, dest, ws = _make_dispatch(t, d, t * k)(x_flat, ew, wrw, offs16, wpack)

    ys = _gmm_call(xs, ws, r_w1, r_w3, r_w2, meta, nwi, te)

    out = _make_combine(t, d, t * k)(ys, shared, dest)

    return (out.reshape(bsz, seq, d), lbl,
            jnp.float32(0.0), jnp.float32(0.0))
